# Initial kernel scaffold; baseline (speedup 1.0000x reference)
import jax
import jax.numpy as jnp
from jax.experimental import pallas as pl


def kernel(x, edge_index, edge_attr, batch, params):
    raise NotImplementedError("write your pallas kernel here")


# speedup vs baseline: 2.7898x; 2.7898x over previous
import functools

import jax
import jax.numpy as jnp
from jax import lax
from jax.experimental import pallas as pl
from jax.experimental.pallas import tpu as pltpu
from jax.experimental.pallas import tpu_sc as plsc

N = 10000
E = 320000
D = 128
DE = 16
G = 64
H = 128

NC = 2
NS = 16
NW = NC * NS

EB = 128
NBLK = E // EB
BLK_PER_W = -(-NBLK // NW)
ROW_STRIDE = 624
ROW_COPY = 640


_EBLK = 8000


def _edge_lin_body(attr_ref, w_ref, b_ref, o1_ref, o2_ref, o3_ref):
    h = jnp.dot(attr_ref[...], w_ref[...], preferred_element_type=jnp.float32)
    h = h + b_ref[...]
    o1_ref[...] = h[:, 0 * H:1 * H]
    o2_ref[...] = h[:, 1 * H:2 * H]
    o3_ref[...] = h[:, 2 * H:3 * H]


def _edge_linears(edge_attr, w_cat, b_cat):
    grid = (E // _EBLK,)
    out = pl.pallas_call(
        _edge_lin_body,
        grid=grid,
        in_specs=[
            pl.BlockSpec((_EBLK, DE), lambda i: (i, 0)),
            pl.BlockSpec((DE, 3 * H), lambda i: (0, 0)),
            pl.BlockSpec((1, 3 * H), lambda i: (0, 0)),
        ],
        out_specs=[
            pl.BlockSpec((_EBLK, H), lambda i: (i, 0)),
            pl.BlockSpec((_EBLK, H), lambda i: (i, 0)),
            pl.BlockSpec((_EBLK, H), lambda i: (i, 0)),
        ],
        out_shape=[jax.ShapeDtypeStruct((E, H), jnp.float32)] * 3,
    )(edge_attr, w_cat, b_cat)
    return out


_sc_mesh = plsc.VectorSubcoreMesh(
    core_axis_name="c", subcore_axis_name="s", num_cores=NC, num_subcores=NS)


@functools.partial(
    pl.kernel,
    out_type=jax.ShapeDtypeStruct((NC, N, D), jnp.float32),
    mesh=_sc_mesh,
    scratch_types=[
        pltpu.VMEM((EB,), jnp.int32),
        pltpu.VMEM((EB,), jnp.int32),
        pltpu.VMEM((EB, D), jnp.float32),
        pltpu.VMEM((EB, D), jnp.float32),
        pltpu.VMEM_SHARED((N, D), jnp.float32),
        pltpu.SemaphoreType.DMA,
    ],
)
def _sc_agg(x_hbm, src_hbm, dst_hbm, e_hbm, zero_hbm, out_hbm,
            src_v, dst_v, xg_v, ev_v, agg_sh, sem):
    cid = lax.axis_index("c")
    sid = lax.axis_index("s")
    wid = sid * NC + cid
    row0 = sid * ROW_STRIDE

    pltpu.sync_copy(zero_hbm.at[pl.ds(row0, ROW_COPY), :],
                    agg_sh.at[pl.ds(row0, ROW_COPY), :])
    plsc.subcore_barrier()

    def block_body(i, carry):
        b = wid + i * NW

        @pl.when(b < NBLK)
        def _():
            base = b * EB
            pltpu.sync_copy(src_hbm.at[pl.ds(base, EB)], src_v)
            pltpu.sync_copy(dst_hbm.at[pl.ds(base, EB)], dst_v)
            pltpu.sync_copy(e_hbm.at[pl.ds(base, EB), :], ev_v)
            pltpu.async_copy(x_hbm.at[src_v], xg_v, sem).wait()

            def row_body(r, c2):
                for cc in range(D // 16):
                    sl = pl.ds(cc * 16, 16)
                    ev_v[r, sl] = jnp.maximum(xg_v[r, sl] + ev_v[r, sl], 0.0)
                return c2

            lax.fori_loop(0, EB, row_body, 0)
            pltpu.sync_copy(ev_v, agg_sh.at[dst_v], add=True)

        return carry

    lax.fori_loop(0, BLK_PER_W, block_body, 0)
    plsc.subcore_barrier()
    pltpu.sync_copy(agg_sh.at[pl.ds(row0, ROW_COPY), :],
                    out_hbm.at[cid, pl.ds(row0, ROW_COPY), :])


def _bn_cols(h, g, b):
    mu = jnp.mean(h, axis=0, keepdims=True)
    xc = h - mu
    var = jnp.mean(xc * xc, axis=0, keepdims=True)
    return g * xc * lax.rsqrt(var + 1e-5) + b


def _node_mlp_body(agg_ref, x_ref, eps_ref, w1_ref, b1_ref, g1_ref, bb1_ref,
                   w2_ref, b2_ref, g2_ref, bb2_ref, go_ref, bo_ref, out_ref):
    agg = agg_ref[0] + agg_ref[1]
    h0 = agg + (1.0 + eps_ref[0, 0]) * x_ref[...]
    h = jnp.dot(h0, w1_ref[...], preferred_element_type=jnp.float32) + b1_ref[...]
    h = _bn_cols(h, g1_ref[...], bb1_ref[...])
    h = jnp.maximum(h, 0.0)
    h = jnp.dot(h, w2_ref[...], preferred_element_type=jnp.float32) + b2_ref[...]
    h = jnp.maximum(h, 0.0)
    h = _bn_cols(h, g2_ref[...], bb2_ref[...])
    h = _bn_cols(h, go_ref[...], bo_ref[...])
    out_ref[...] = h


def _node_mlp(agg, x, eps, w1t, b1, g1, bb1, w2t, b2, g2, bb2, go, bo):
    return pl.pallas_call(
        _node_mlp_body,
        out_shape=jax.ShapeDtypeStruct((N, H), jnp.float32),
    )(agg, x, eps, w1t, b1, g1, bb1, w2t, b2, g2, bb2, go, bo)


def _readout_body(x1_ref, x2_ref, x3_ref, batch_ref, w1_ref, b1_ref,
                  g_ref, bb_ref, w2_ref, b2_ref, out_ref):
    bvec = batch_ref[...]
    gids = lax.broadcasted_iota(jnp.int32, (G, N), 0)
    onehot = (gids == bvec).astype(jnp.float32)
    p1 = jnp.dot(onehot, x1_ref[...], preferred_element_type=jnp.float32)
    p2 = jnp.dot(onehot, x2_ref[...], preferred_element_type=jnp.float32)
    p3 = jnp.dot(onehot, x3_ref[...], preferred_element_type=jnp.float32)
    h = jnp.concatenate([p1, p2, p3], axis=1)
    h = jnp.dot(h, w1_ref[...], preferred_element_type=jnp.float32) + b1_ref[...]
    h = _bn_cols(h, g_ref[...], bb_ref[...])
    h = jnp.where(h >= 0.0, h, 0.01 * h)
    out_ref[...] = (
        jnp.dot(h, w2_ref[...], preferred_element_type=jnp.float32) + b2_ref[...])


def _readout(x1, x2, x3, batch2d, w1t, b1, g, bb, w2t, b2):
    return pl.pallas_call(
        _readout_body,
        out_shape=jax.ShapeDtypeStruct((G, 1), jnp.float32),
    )(x1, x2, x3, batch2d, w1t, b1, g, bb, w2t, b2)


def kernel(x, edge_index, edge_attr, batch, params):
    p = params
    src = edge_index[0].astype(jnp.int32)
    dst = edge_index[1].astype(jnp.int32)

    w_cat = jnp.concatenate(
        [p['g1_lin_W'].T, p['g2_lin_W'].T, p['g3_lin_W'].T], axis=1)
    b_cat = jnp.concatenate(
        [p['g1_lin_b'], p['g2_lin_b'], p['g3_lin_b']]).reshape(1, 3 * H)
    e1, e2, e3 = _edge_linears(edge_attr, w_cat, b_cat)

    zeros = jnp.zeros((N, D), jnp.float32)

    def row(v):
        return v.reshape(1, -1)

    h = x
    feats = []
    for pre, e_l, og, ob in (
            ('g1_', e1, p['bn1_g'], p['bn1_b']),
            ('g2_', e2, p['bn2_g'], p['bn2_b']),
            ('g3_', e3, p['bn3_g'], p['bn3_b'])):
        agg = _sc_agg(h, src, dst, e_l, zeros)
        h = _node_mlp(
            agg, h, p[pre + 'eps'].reshape(1, 1).astype(jnp.float32),
            p[pre + 'W1'].T, row(p[pre + 'b1']),
            row(p[pre + 'bn1_g']), row(p[pre + 'bn1_b']),
            p[pre + 'W2'].T, row(p[pre + 'b2']),
            row(p[pre + 'bn2_g']), row(p[pre + 'bn2_b']),
            row(og), row(ob))
        feats.append(h)

    x1, x2, x3 = feats
    return _readout(
        x1, x2, x3, batch.astype(jnp.int32).reshape(1, N),
        p['lin1_W'].T, row(p['lin1_b']),
        row(p['bn4_g']), row(p['bn4_b']),
        p['lin2_W'].T, row(p['lin2_b']))


# speedup vs baseline: 4.8801x; 1.7492x over previous
import functools

import jax
import jax.numpy as jnp
from jax import lax
from jax.experimental import pallas as pl
from jax.experimental.pallas import tpu as pltpu
from jax.experimental.pallas import tpu_sc as plsc

N = 10000
E = 320000
D = 128
DE = 16
G = 64
H = 128

NC = 2
NS = 16
NW = NC * NS

EB = 64
NBLK = E // EB
BLK_LO = NBLK // NW
BLK_EXTRA = NBLK % NW
GROUP = 3
GROUPS = BLK_LO // GROUP
assert GROUPS * GROUP == BLK_LO
ROW_STRIDE = 624
ROW_COPY = 640


_EBLK = 8000


def _edge_lin_body(attr_ref, w_ref, b_ref, o1_ref, o2_ref, o3_ref):
    h = jnp.dot(attr_ref[...], w_ref[...], preferred_element_type=jnp.float32)
    h = h + b_ref[...]
    o1_ref[...] = h[:, 0 * H:1 * H]
    o2_ref[...] = h[:, 1 * H:2 * H]
    o3_ref[...] = h[:, 2 * H:3 * H]


def _edge_linears(edge_attr, w_cat, b_cat):
    grid = (E // _EBLK,)
    out = pl.pallas_call(
        _edge_lin_body,
        grid=grid,
        in_specs=[
            pl.BlockSpec((_EBLK, DE), lambda i: (i, 0)),
            pl.BlockSpec((DE, 3 * H), lambda i: (0, 0)),
            pl.BlockSpec((1, 3 * H), lambda i: (0, 0)),
        ],
        out_specs=[
            pl.BlockSpec((_EBLK, H), lambda i: (i, 0)),
            pl.BlockSpec((_EBLK, H), lambda i: (i, 0)),
            pl.BlockSpec((_EBLK, H), lambda i: (i, 0)),
        ],
        out_shape=[jax.ShapeDtypeStruct((E, H), jnp.float32)] * 3,
    )(edge_attr, w_cat, b_cat)
    return out


_sc_mesh = plsc.VectorSubcoreMesh(
    core_axis_name="c", subcore_axis_name="s", num_cores=NC, num_subcores=NS)


@functools.partial(
    pl.kernel,
    out_type=jax.ShapeDtypeStruct((NC, N, D), jnp.float32),
    mesh=_sc_mesh,
    scratch_types=[
        pltpu.VMEM((EB,), jnp.int32),
        pltpu.VMEM((EB,), jnp.int32),
        pltpu.VMEM((EB,), jnp.int32),
        pltpu.VMEM((EB,), jnp.int32),
        pltpu.VMEM((EB,), jnp.int32),
        pltpu.VMEM((EB,), jnp.int32),
        pltpu.VMEM((EB, D), jnp.float32),
        pltpu.VMEM((EB, D), jnp.float32),
        pltpu.VMEM((EB, D), jnp.float32),
        pltpu.VMEM((EB, D), jnp.float32),
        pltpu.VMEM((EB, D), jnp.float32),
        pltpu.VMEM((EB, D), jnp.float32),
        pltpu.VMEM_SHARED((N, D), jnp.float32),
        pltpu.SemaphoreType.DMA,
        pltpu.SemaphoreType.DMA,
        pltpu.SemaphoreType.DMA,
        pltpu.SemaphoreType.DMA,
        pltpu.SemaphoreType.DMA,
        pltpu.SemaphoreType.DMA,
        pltpu.SemaphoreType.DMA,
        pltpu.SemaphoreType.DMA,
        pltpu.SemaphoreType.DMA,
        pltpu.SemaphoreType.DMA,
        pltpu.SemaphoreType.DMA,
        pltpu.SemaphoreType.DMA,
    ],
)
def _sc_agg(x_hbm, src_hbm, dst_hbm, e_hbm, zero_hbm, out_hbm,
            si0, si1, si2, di0, di1, di2, xg0, xg1, xg2, ev0, ev1, ev2,
            agg_sh, is0, is1, is2, gs0, gs1, gs2, es0, es1, es2,
            ss0, ss1, ss2):
    cid = lax.axis_index("c")
    sid = lax.axis_index("s")
    wid = sid * NC + cid
    row0 = sid * ROW_STRIDE
    nb = BLK_LO + jnp.where(wid < BLK_EXTRA, 1, 0)

    sis = [si0, si1, si2]
    dis = [di0, di1, di2]
    xgs = [xg0, xg1, xg2]
    evs = [ev0, ev1, ev2]
    isems = [is0, is1, is2]
    gsems = [gs0, gs1, gs2]
    esems = [es0, es1, es2]
    ssems = [ss0, ss1, ss2]

    def idx_start(i, sl):
        base = (wid + i * NW) * EB
        pltpu.make_async_copy(src_hbm.at[pl.ds(base, EB)], sis[sl],
                              isems[sl]).start()
        pltpu.make_async_copy(dst_hbm.at[pl.ds(base, EB)], dis[sl],
                              isems[sl]).start()

    def idx_wait(sl):
        pltpu.make_async_copy(src_hbm.at[pl.ds(0, EB)], sis[sl],
                              isems[sl]).wait()
        pltpu.make_async_copy(dst_hbm.at[pl.ds(0, EB)], dis[sl],
                              isems[sl]).wait()

    def e_start(i, sl):
        pltpu.make_async_copy(e_hbm.at[pl.ds((wid + i * NW) * EB, EB), :],
                              evs[sl], esems[sl]).start()

    def e_wait(sl):
        pltpu.make_async_copy(e_hbm.at[pl.ds(0, EB), :], evs[sl],
                              esems[sl]).wait()

    def g_start(sl_x, sl_i):
        pltpu.make_async_copy(x_hbm.at[sis[sl_i]], xgs[sl_x],
                              gsems[sl_x]).start()

    def g_wait(sl_x, sl_i):
        pltpu.make_async_copy(x_hbm.at[sis[sl_i]], xgs[sl_x],
                              gsems[sl_x]).wait()

    def s_start(sl):
        pltpu.async_copy(evs[sl], agg_sh.at[dis[sl]], ssems[sl], add=True)

    def s_wait(sl):
        pltpu.make_async_copy(evs[sl], agg_sh.at[dis[sl]], ssems[sl]).wait()

    def compute(sl_x, sl_e):
        xg = xgs[sl_x]
        ev = evs[sl_e]

        def row_body(r, c2):
            for rr in range(2):
                for cc in range(D // 16):
                    sl = pl.ds(cc * 16, 16)
                    ev[2 * r + rr, sl] = jnp.maximum(
                        xg[2 * r + rr, sl] + ev[2 * r + rr, sl], 0.0)
            return c2

        lax.fori_loop(0, EB // 2, row_body, 0)

    pltpu.sync_copy(zero_hbm.at[pl.ds(row0, ROW_COPY), :],
                    agg_sh.at[pl.ds(row0, ROW_COPY), :])
    plsc.subcore_barrier()

    idx_start(0, 0)
    idx_start(1, 1)
    idx_wait(0)
    g_start(0, 0)
    e_start(0, 0)
    e_start(1, 1)

    def group_body(g, carry):
        for jj in range(GROUP):
            i = g * GROUP + jj
            sl3 = jj % 3

            @pl.when(i >= 1)
            def _():
                s_wait((jj + 2) % 3)

            @pl.when(i + 2 < nb)
            def _():
                idx_start(i + 2, (jj + 2) % 3)
                e_start(i + 2, (jj + 2) % 3)

            @pl.when(i + 1 < nb)
            def _():
                idx_wait((jj + 1) % 3)
                g_start((jj + 1) % 3, (jj + 1) % 3)

            g_wait(sl3, sl3)
            e_wait(sl3)
            compute(sl3, sl3)
            s_start(sl3)
        return carry

    lax.fori_loop(0, GROUPS, group_body, 0)

    @pl.when(nb > BLK_LO)
    def _():
        s_wait(2)
        g_wait(0, 0)
        e_wait(0)
        compute(0, 0)
        s_start(0)
        s_wait(0)

    @pl.when(nb == BLK_LO)
    def _():
        s_wait(2)

    plsc.subcore_barrier()
    pltpu.sync_copy(agg_sh.at[pl.ds(row0, ROW_COPY), :],
                    out_hbm.at[cid, pl.ds(row0, ROW_COPY), :])


def _bn_cols(h, g, b):
    mu = jnp.mean(h, axis=0, keepdims=True)
    xc = h - mu
    var = jnp.mean(xc * xc, axis=0, keepdims=True)
    return g * xc * lax.rsqrt(var + 1e-5) + b


def _node_mlp_body(agg_ref, x_ref, eps_ref, w1_ref, b1_ref, g1_ref, bb1_ref,
                   w2_ref, b2_ref, g2_ref, bb2_ref, go_ref, bo_ref, out_ref):
    agg = agg_ref[0] + agg_ref[1]
    h0 = agg + (1.0 + eps_ref[0, 0]) * x_ref[...]
    h = jnp.dot(h0, w1_ref[...], preferred_element_type=jnp.float32) + b1_ref[...]
    h = _bn_cols(h, g1_ref[...], bb1_ref[...])
    h = jnp.maximum(h, 0.0)
    h = jnp.dot(h, w2_ref[...], preferred_element_type=jnp.float32) + b2_ref[...]
    h = jnp.maximum(h, 0.0)
    h = _bn_cols(h, g2_ref[...], bb2_ref[...])
    h = _bn_cols(h, go_ref[...], bo_ref[...])
    out_ref[...] = h


def _node_mlp(agg, x, eps, w1t, b1, g1, bb1, w2t, b2, g2, bb2, go, bo):
    return pl.pallas_call(
        _node_mlp_body,
        out_shape=jax.ShapeDtypeStruct((N, H), jnp.float32),
    )(agg, x, eps, w1t, b1, g1, bb1, w2t, b2, g2, bb2, go, bo)


def _readout_body(x1_ref, x2_ref, x3_ref, batch_ref, w1_ref, b1_ref,
                  g_ref, bb_ref, w2_ref, b2_ref, out_ref):
    bvec = batch_ref[...]
    gids = lax.broadcasted_iota(jnp.int32, (G, N), 0)
    onehot = (gids == bvec).astype(jnp.float32)
    p1 = jnp.dot(onehot, x1_ref[...], preferred_element_type=jnp.float32)
    p2 = jnp.dot(onehot, x2_ref[...], preferred_element_type=jnp.float32)
    p3 = jnp.dot(onehot, x3_ref[...], preferred_element_type=jnp.float32)
    h = jnp.concatenate([p1, p2, p3], axis=1)
    h = jnp.dot(h, w1_ref[...], preferred_element_type=jnp.float32) + b1_ref[...]
    h = _bn_cols(h, g_ref[...], bb_ref[...])
    h = jnp.where(h >= 0.0, h, 0.01 * h)
    out_ref[...] = (
        jnp.dot(h, w2_ref[...], preferred_element_type=jnp.float32) + b2_ref[...])


def _readout(x1, x2, x3, batch2d, w1t, b1, g, bb, w2t, b2):
    return pl.pallas_call(
        _readout_body,
        out_shape=jax.ShapeDtypeStruct((G, 1), jnp.float32),
    )(x1, x2, x3, batch2d, w1t, b1, g, bb, w2t, b2)


def kernel(x, edge_index, edge_attr, batch, params):
    p = params
    src = edge_index[0].astype(jnp.int32)
    dst = edge_index[1].astype(jnp.int32)

    w_cat = jnp.concatenate(
        [p['g1_lin_W'].T, p['g2_lin_W'].T, p['g3_lin_W'].T], axis=1)
    b_cat = jnp.concatenate(
        [p['g1_lin_b'], p['g2_lin_b'], p['g3_lin_b']]).reshape(1, 3 * H)
    e1, e2, e3 = _edge_linears(edge_attr, w_cat, b_cat)

    zeros = jnp.zeros((N, D), jnp.float32)

    def row(v):
        return v.reshape(1, -1)

    h = x
    feats = []
    for pre, e_l, og, ob in (
            ('g1_', e1, p['bn1_g'], p['bn1_b']),
            ('g2_', e2, p['bn2_g'], p['bn2_b']),
            ('g3_', e3, p['bn3_g'], p['bn3_b'])):
        agg = _sc_agg(h, src, dst, e_l, zeros)
        h = _node_mlp(
            agg, h, p[pre + 'eps'].reshape(1, 1).astype(jnp.float32),
            p[pre + 'W1'].T, row(p[pre + 'b1']),
            row(p[pre + 'bn1_g']), row(p[pre + 'bn1_b']),
            p[pre + 'W2'].T, row(p[pre + 'b2']),
            row(p[pre + 'bn2_g']), row(p[pre + 'bn2_b']),
            row(og), row(ob))
        feats.append(h)

    x1, x2, x3 = feats
    return _readout(
        x1, x2, x3, batch.astype(jnp.int32).reshape(1, N),
        p['lin1_W'].T, row(p['lin1_b']),
        row(p['bn4_g']), row(p['bn4_b']),
        p['lin2_W'].T, row(p['lin2_b']))


# speedup vs baseline: 4.8938x; 1.0028x over previous
import functools

import jax
import jax.numpy as jnp
from jax import lax
from jax.experimental import pallas as pl
from jax.experimental.pallas import tpu as pltpu
from jax.experimental.pallas import tpu_sc as plsc

N = 10000
E = 320000
D = 128
DE = 16
G = 64
H = 128

NC = 2
NS = 16
NW = NC * NS

EB = 64
NBLK = E // EB
BLK_LO = NBLK // NW
BLK_EXTRA = NBLK % NW
GROUP = 3
GROUPS = BLK_LO // GROUP
assert GROUPS * GROUP == BLK_LO
ROW_STRIDE = 624
ROW_COPY = 640


_EBLK = 8000


def _edge_lin_body(attr_ref, w_ref, b_ref, o1_ref, o2_ref, o3_ref):
    h = jnp.dot(attr_ref[...], w_ref[...], preferred_element_type=jnp.float32)
    h = h + b_ref[...]
    o1_ref[...] = h[:, 0 * H:1 * H]
    o2_ref[...] = h[:, 1 * H:2 * H]
    o3_ref[...] = h[:, 2 * H:3 * H]


def _edge_linears(edge_attr, w_cat, b_cat):
    grid = (E // _EBLK,)
    out = pl.pallas_call(
        _edge_lin_body,
        grid=grid,
        in_specs=[
            pl.BlockSpec((_EBLK, DE), lambda i: (i, 0)),
            pl.BlockSpec((DE, 3 * H), lambda i: (0, 0)),
            pl.BlockSpec((1, 3 * H), lambda i: (0, 0)),
        ],
        out_specs=[
            pl.BlockSpec((_EBLK, H), lambda i: (i, 0)),
            pl.BlockSpec((_EBLK, H), lambda i: (i, 0)),
            pl.BlockSpec((_EBLK, H), lambda i: (i, 0)),
        ],
        out_shape=[jax.ShapeDtypeStruct((E, H), jnp.float32)] * 3,
    )(edge_attr, w_cat, b_cat)
    return out


_sc_mesh = plsc.VectorSubcoreMesh(
    core_axis_name="c", subcore_axis_name="s", num_cores=NC, num_subcores=NS)


@functools.partial(
    pl.kernel,
    out_type=jax.ShapeDtypeStruct((NC, N, D), jnp.float32),
    mesh=_sc_mesh,
    compiler_params=pltpu.CompilerParams(use_tc_tiling_on_sc=True),
    scratch_types=[
        pltpu.VMEM((EB,), jnp.int32),
        pltpu.VMEM((EB,), jnp.int32),
        pltpu.VMEM((EB,), jnp.int32),
        pltpu.VMEM((EB,), jnp.int32),
        pltpu.VMEM((EB,), jnp.int32),
        pltpu.VMEM((EB,), jnp.int32),
        pltpu.VMEM((EB, D), jnp.float32),
        pltpu.VMEM((EB, D), jnp.float32),
        pltpu.VMEM((EB, D), jnp.float32),
        pltpu.VMEM((EB, D), jnp.float32),
        pltpu.VMEM((EB, D), jnp.float32),
        pltpu.VMEM((EB, D), jnp.float32),
        pltpu.VMEM_SHARED((N, D), jnp.float32),
        pltpu.SemaphoreType.DMA,
        pltpu.SemaphoreType.DMA,
        pltpu.SemaphoreType.DMA,
        pltpu.SemaphoreType.DMA,
        pltpu.SemaphoreType.DMA,
        pltpu.SemaphoreType.DMA,
        pltpu.SemaphoreType.DMA,
        pltpu.SemaphoreType.DMA,
        pltpu.SemaphoreType.DMA,
        pltpu.SemaphoreType.DMA,
        pltpu.SemaphoreType.DMA,
        pltpu.SemaphoreType.DMA,
    ],
)
def _sc_agg(x_hbm, src_hbm, dst_hbm, e_hbm, zero_hbm, out_hbm,
            si0, si1, si2, di0, di1, di2, xg0, xg1, xg2, ev0, ev1, ev2,
            agg_sh, is0, is1, is2, gs0, gs1, gs2, es0, es1, es2,
            ss0, ss1, ss2):
    cid = lax.axis_index("c")
    sid = lax.axis_index("s")
    wid = sid * NC + cid
    row0 = sid * ROW_STRIDE
    nb = BLK_LO + jnp.where(wid < BLK_EXTRA, 1, 0)

    sis = [si0, si1, si2]
    dis = [di0, di1, di2]
    xgs = [xg0, xg1, xg2]
    evs = [ev0, ev1, ev2]
    isems = [is0, is1, is2]
    gsems = [gs0, gs1, gs2]
    esems = [es0, es1, es2]
    ssems = [ss0, ss1, ss2]

    def idx_start(i, sl):
        base = (wid + i * NW) * EB
        pltpu.make_async_copy(src_hbm.at[pl.ds(base, EB)], sis[sl],
                              isems[sl]).start()
        pltpu.make_async_copy(dst_hbm.at[pl.ds(base, EB)], dis[sl],
                              isems[sl]).start()

    def idx_wait(sl):
        pltpu.make_async_copy(src_hbm.at[pl.ds(0, EB)], sis[sl],
                              isems[sl]).wait()
        pltpu.make_async_copy(dst_hbm.at[pl.ds(0, EB)], dis[sl],
                              isems[sl]).wait()

    def e_start(i, sl):
        pltpu.make_async_copy(e_hbm.at[pl.ds((wid + i * NW) * EB, EB), :],
                              evs[sl], esems[sl]).start()

    def e_wait(sl):
        pltpu.make_async_copy(e_hbm.at[pl.ds(0, EB), :], evs[sl],
                              esems[sl]).wait()

    def g_start(sl_x, sl_i):
        pltpu.make_async_copy(x_hbm.at[sis[sl_i]], xgs[sl_x],
                              gsems[sl_x]).start()

    def g_wait(sl_x, sl_i):
        pltpu.make_async_copy(x_hbm.at[sis[sl_i]], xgs[sl_x],
                              gsems[sl_x]).wait()

    def s_start(sl):
        pltpu.async_copy(evs[sl], agg_sh.at[dis[sl]], ssems[sl], add=True)

    def s_wait(sl):
        pltpu.make_async_copy(evs[sl], agg_sh.at[dis[sl]], ssems[sl]).wait()

    def compute(sl_x, sl_e):
        xg = xgs[sl_x]
        ev = evs[sl_e]

        def row_body(r, c2):
            for rr in range(2):
                for cc in range(D // 16):
                    sl = pl.ds(cc * 16, 16)
                    ev[2 * r + rr, sl] = jnp.maximum(
                        xg[2 * r + rr, sl] + ev[2 * r + rr, sl], 0.0)
            return c2

        lax.fori_loop(0, EB // 2, row_body, 0)

    pltpu.sync_copy(zero_hbm.at[pl.ds(row0, ROW_COPY), :],
                    agg_sh.at[pl.ds(row0, ROW_COPY), :])
    plsc.subcore_barrier()

    idx_start(0, 0)
    idx_start(1, 1)
    idx_wait(0)
    g_start(0, 0)
    e_start(0, 0)
    e_start(1, 1)

    def group_body(g, carry):
        for jj in range(GROUP):
            i = g * GROUP + jj
            sl3 = jj % 3

            @pl.when(i >= 1)
            def _():
                s_wait((jj + 2) % 3)

            @pl.when(i + 2 < nb)
            def _():
                idx_start(i + 2, (jj + 2) % 3)
                e_start(i + 2, (jj + 2) % 3)

            @pl.when(i + 1 < nb)
            def _():
                idx_wait((jj + 1) % 3)
                g_start((jj + 1) % 3, (jj + 1) % 3)

            g_wait(sl3, sl3)
            e_wait(sl3)
            compute(sl3, sl3)
            s_start(sl3)
        return carry

    lax.fori_loop(0, GROUPS, group_body, 0)

    @pl.when(nb > BLK_LO)
    def _():
        s_wait(2)
        g_wait(0, 0)
        e_wait(0)
        compute(0, 0)
        s_start(0)
        s_wait(0)

    @pl.when(nb == BLK_LO)
    def _():
        s_wait(2)

    plsc.subcore_barrier()
    pltpu.sync_copy(agg_sh.at[pl.ds(row0, ROW_COPY), :],
                    out_hbm.at[cid, pl.ds(row0, ROW_COPY), :])


def _bn_cols(h, g, b):
    mu = jnp.mean(h, axis=0, keepdims=True)
    xc = h - mu
    var = jnp.mean(xc * xc, axis=0, keepdims=True)
    return g * xc * lax.rsqrt(var + 1e-5) + b


def _node_mlp_body(agg_ref, x_ref, eps_ref, w1_ref, b1_ref, g1_ref, bb1_ref,
                   w2_ref, b2_ref, g2_ref, bb2_ref, go_ref, bo_ref, out_ref):
    agg = agg_ref[0] + agg_ref[1]
    h0 = agg + (1.0 + eps_ref[0, 0]) * x_ref[...]
    h = jnp.dot(h0, w1_ref[...], preferred_element_type=jnp.float32) + b1_ref[...]
    h = _bn_cols(h, g1_ref[...], bb1_ref[...])
    h = jnp.maximum(h, 0.0)
    h = jnp.dot(h, w2_ref[...], preferred_element_type=jnp.float32) + b2_ref[...]
    h = jnp.maximum(h, 0.0)
    h = _bn_cols(h, g2_ref[...], bb2_ref[...])
    h = _bn_cols(h, go_ref[...], bo_ref[...])
    out_ref[...] = h


def _node_mlp(agg, x, eps, w1t, b1, g1, bb1, w2t, b2, g2, bb2, go, bo):
    return pl.pallas_call(
        _node_mlp_body,
        out_shape=jax.ShapeDtypeStruct((N, H), jnp.float32),
    )(agg, x, eps, w1t, b1, g1, bb1, w2t, b2, g2, bb2, go, bo)


def _readout_body(x1_ref, x2_ref, x3_ref, batch_ref, w1_ref, b1_ref,
                  g_ref, bb_ref, w2_ref, b2_ref, out_ref):
    bvec = batch_ref[...]
    gids = lax.broadcasted_iota(jnp.int32, (G, N), 0)
    onehot = (gids == bvec).astype(jnp.float32)
    p1 = jnp.dot(onehot, x1_ref[...], preferred_element_type=jnp.float32)
    p2 = jnp.dot(onehot, x2_ref[...], preferred_element_type=jnp.float32)
    p3 = jnp.dot(onehot, x3_ref[...], preferred_element_type=jnp.float32)
    h = jnp.concatenate([p1, p2, p3], axis=1)
    h = jnp.dot(h, w1_ref[...], preferred_element_type=jnp.float32) + b1_ref[...]
    h = _bn_cols(h, g_ref[...], bb_ref[...])
    h = jnp.where(h >= 0.0, h, 0.01 * h)
    out_ref[...] = (
        jnp.dot(h, w2_ref[...], preferred_element_type=jnp.float32) + b2_ref[...])


def _readout(x1, x2, x3, batch2d, w1t, b1, g, bb, w2t, b2):
    return pl.pallas_call(
        _readout_body,
        out_shape=jax.ShapeDtypeStruct((G, 1), jnp.float32),
    )(x1, x2, x3, batch2d, w1t, b1, g, bb, w2t, b2)


def kernel(x, edge_index, edge_attr, batch, params):
    p = params
    src = edge_index[0].astype(jnp.int32)
    dst = edge_index[1].astype(jnp.int32)

    w_cat = jnp.concatenate(
        [p['g1_lin_W'].T, p['g2_lin_W'].T, p['g3_lin_W'].T], axis=1)
    b_cat = jnp.concatenate(
        [p['g1_lin_b'], p['g2_lin_b'], p['g3_lin_b']]).reshape(1, 3 * H)
    e1, e2, e3 = _edge_linears(edge_attr, w_cat, b_cat)

    zeros = jnp.zeros((N, D), jnp.float32)

    def row(v):
        return v.reshape(1, -1)

    h = x
    feats = []
    for pre, e_l, og, ob in (
            ('g1_', e1, p['bn1_g'], p['bn1_b']),
            ('g2_', e2, p['bn2_g'], p['bn2_b']),
            ('g3_', e3, p['bn3_g'], p['bn3_b'])):
        agg = _sc_agg(h, src, dst, e_l, zeros)
        h = _node_mlp(
            agg, h, p[pre + 'eps'].reshape(1, 1).astype(jnp.float32),
            p[pre + 'W1'].T, row(p[pre + 'b1']),
            row(p[pre + 'bn1_g']), row(p[pre + 'bn1_b']),
            p[pre + 'W2'].T, row(p[pre + 'b2']),
            row(p[pre + 'bn2_g']), row(p[pre + 'bn2_b']),
            row(og), row(ob))
        feats.append(h)

    x1, x2, x3 = feats
    return _readout(
        x1, x2, x3, batch.astype(jnp.int32).reshape(1, N),
        p['lin1_W'].T, row(p['lin1_b']),
        row(p['bn4_g']), row(p['bn4_b']),
        p['lin2_W'].T, row(p['lin2_b']))


# speedup vs baseline: 5.6516x; 1.1548x over previous
import functools

import jax
import jax.numpy as jnp
from jax import lax
from jax.experimental import pallas as pl
from jax.experimental.pallas import tpu as pltpu
from jax.experimental.pallas import tpu_sc as plsc

N = 10000
E = 320000
D = 128
DE = 16
G = 64
H = 128

NC = 2
NS = 16
NW = NC * NS

EB = 64
NBLK = E // EB
BLK_LO = NBLK // NW
BLK_EXTRA = NBLK % NW
GROUP = 3
GROUPS = BLK_LO // GROUP
assert GROUPS * GROUP == BLK_LO
ROW_STRIDE = 624
ROW_COPY = 640


_EBLK = 6400


def _edge_lin_body(attr_ref, w_ref, b_ref, o1_ref, o2_ref, o3_ref):
    h = lax.dot_general(attr_ref[...], w_ref[...],
                        dimension_numbers=(((0,), (0,)), ((), ())),
                        preferred_element_type=jnp.float32)
    h = h + b_ref[...]
    o1_ref[...] = h[:, 0 * H:1 * H]
    o2_ref[...] = h[:, 1 * H:2 * H]
    o3_ref[...] = h[:, 2 * H:3 * H]


def _edge_linears(attr_t, w_cat, b_cat):
    grid = (E // _EBLK,)
    out = pl.pallas_call(
        _edge_lin_body,
        grid=grid,
        in_specs=[
            pl.BlockSpec((DE, _EBLK), lambda i: (0, i)),
            pl.BlockSpec((DE, 3 * H), lambda i: (0, 0)),
            pl.BlockSpec((1, 3 * H), lambda i: (0, 0)),
        ],
        out_specs=[
            pl.BlockSpec((_EBLK, H), lambda i: (i, 0)),
            pl.BlockSpec((_EBLK, H), lambda i: (i, 0)),
            pl.BlockSpec((_EBLK, H), lambda i: (i, 0)),
        ],
        out_shape=[jax.ShapeDtypeStruct((E, H), jnp.float32)] * 3,
    )(attr_t, w_cat, b_cat)
    return out


_sc_mesh = plsc.VectorSubcoreMesh(
    core_axis_name="c", subcore_axis_name="s", num_cores=NC, num_subcores=NS)


@functools.partial(
    pl.kernel,
    out_type=jax.ShapeDtypeStruct((NC, N, D), jnp.float32),
    mesh=_sc_mesh,
    compiler_params=pltpu.CompilerParams(use_tc_tiling_on_sc=True),
    scratch_types=[
        pltpu.VMEM((EB,), jnp.int32),
        pltpu.VMEM((EB,), jnp.int32),
        pltpu.VMEM((EB,), jnp.int32),
        pltpu.VMEM((EB,), jnp.int32),
        pltpu.VMEM((EB,), jnp.int32),
        pltpu.VMEM((EB,), jnp.int32),
        pltpu.VMEM((EB, D), jnp.float32),
        pltpu.VMEM((EB, D), jnp.float32),
        pltpu.VMEM((EB, D), jnp.float32),
        pltpu.VMEM((EB, D), jnp.float32),
        pltpu.VMEM((EB, D), jnp.float32),
        pltpu.VMEM((EB, D), jnp.float32),
        pltpu.VMEM_SHARED((N, D), jnp.float32),
        pltpu.SemaphoreType.DMA,
        pltpu.SemaphoreType.DMA,
        pltpu.SemaphoreType.DMA,
        pltpu.SemaphoreType.DMA,
        pltpu.SemaphoreType.DMA,
        pltpu.SemaphoreType.DMA,
        pltpu.SemaphoreType.DMA,
        pltpu.SemaphoreType.DMA,
        pltpu.SemaphoreType.DMA,
        pltpu.SemaphoreType.DMA,
        pltpu.SemaphoreType.DMA,
        pltpu.SemaphoreType.DMA,
    ],
)
def _sc_agg(x_hbm, src_hbm, dst_hbm, e_hbm, zero_hbm, out_hbm,
            si0, si1, si2, di0, di1, di2, xg0, xg1, xg2, ev0, ev1, ev2,
            agg_sh, is0, is1, is2, gs0, gs1, gs2, es0, es1, es2,
            ss0, ss1, ss2):
    cid = lax.axis_index("c")
    sid = lax.axis_index("s")
    wid = sid * NC + cid
    row0 = sid * ROW_STRIDE
    nb = BLK_LO + jnp.where(wid < BLK_EXTRA, 1, 0)

    sis = [si0, si1, si2]
    dis = [di0, di1, di2]
    xgs = [xg0, xg1, xg2]
    evs = [ev0, ev1, ev2]
    isems = [is0, is1, is2]
    gsems = [gs0, gs1, gs2]
    esems = [es0, es1, es2]
    ssems = [ss0, ss1, ss2]

    def idx_start(i, sl):
        base = (wid + i * NW) * EB
        pltpu.make_async_copy(src_hbm.at[pl.ds(base, EB)], sis[sl],
                              isems[sl]).start()
        pltpu.make_async_copy(dst_hbm.at[pl.ds(base, EB)], dis[sl],
                              isems[sl]).start()

    def idx_wait(sl):
        pltpu.make_async_copy(src_hbm.at[pl.ds(0, EB)], sis[sl],
                              isems[sl]).wait()
        pltpu.make_async_copy(dst_hbm.at[pl.ds(0, EB)], dis[sl],
                              isems[sl]).wait()

    def e_start(i, sl):
        pltpu.make_async_copy(e_hbm.at[pl.ds((wid + i * NW) * EB, EB), :],
                              evs[sl], esems[sl]).start()

    def e_wait(sl):
        pltpu.make_async_copy(e_hbm.at[pl.ds(0, EB), :], evs[sl],
                              esems[sl]).wait()

    def g_start(sl_x, sl_i):
        pltpu.make_async_copy(x_hbm.at[sis[sl_i]], xgs[sl_x],
                              gsems[sl_x]).start()

    def g_wait(sl_x, sl_i):
        pltpu.make_async_copy(x_hbm.at[sis[sl_i]], xgs[sl_x],
                              gsems[sl_x]).wait()

    def s_start(sl):
        pltpu.async_copy(evs[sl], agg_sh.at[dis[sl]], ssems[sl], add=True)

    def s_wait(sl):
        pltpu.make_async_copy(evs[sl], agg_sh.at[dis[sl]], ssems[sl]).wait()

    def compute(sl_x, sl_e):
        xg = xgs[sl_x]
        ev = evs[sl_e]

        def row_body(r, c2):
            for rr in range(2):
                for cc in range(D // 16):
                    sl = pl.ds(cc * 16, 16)
                    ev[2 * r + rr, sl] = jnp.maximum(
                        xg[2 * r + rr, sl] + ev[2 * r + rr, sl], 0.0)
            return c2

        lax.fori_loop(0, EB // 2, row_body, 0)

    pltpu.sync_copy(zero_hbm.at[pl.ds(row0, ROW_COPY), :],
                    agg_sh.at[pl.ds(row0, ROW_COPY), :])
    plsc.subcore_barrier()

    idx_start(0, 0)
    idx_start(1, 1)
    idx_wait(0)
    g_start(0, 0)
    e_start(0, 0)
    e_start(1, 1)

    def group_body(g, carry):
        for jj in range(GROUP):
            i = g * GROUP + jj
            sl3 = jj % 3

            @pl.when(i >= 1)
            def _():
                s_wait((jj + 2) % 3)

            @pl.when(i + 2 < nb)
            def _():
                idx_start(i + 2, (jj + 2) % 3)
                e_start(i + 2, (jj + 2) % 3)

            @pl.when(i + 1 < nb)
            def _():
                idx_wait((jj + 1) % 3)
                g_start((jj + 1) % 3, (jj + 1) % 3)

            g_wait(sl3, sl3)
            e_wait(sl3)
            compute(sl3, sl3)
            s_start(sl3)
        return carry

    lax.fori_loop(0, GROUPS, group_body, 0)

    @pl.when(nb > BLK_LO)
    def _():
        s_wait(2)
        g_wait(0, 0)
        e_wait(0)
        compute(0, 0)
        s_start(0)
        s_wait(0)

    @pl.when(nb == BLK_LO)
    def _():
        s_wait(2)

    plsc.subcore_barrier()
    pltpu.sync_copy(agg_sh.at[pl.ds(row0, ROW_COPY), :],
                    out_hbm.at[cid, pl.ds(row0, ROW_COPY), :])


def _bn_cols(h, g, b):
    mu = jnp.mean(h, axis=0, keepdims=True)
    xc = h - mu
    var = jnp.mean(xc * xc, axis=0, keepdims=True)
    return g * xc * lax.rsqrt(var + 1e-5) + b


def _node_mlp_body(agg_ref, x_ref, eps_ref, w1_ref, b1_ref, g1_ref, bb1_ref,
                   w2_ref, b2_ref, g2_ref, bb2_ref, go_ref, bo_ref, out_ref):
    agg = agg_ref[0] + agg_ref[1]
    h0 = agg + (1.0 + eps_ref[0, 0]) * x_ref[...]
    h = jnp.dot(h0, w1_ref[...], preferred_element_type=jnp.float32) + b1_ref[...]
    h = _bn_cols(h, g1_ref[...], bb1_ref[...])
    h = jnp.maximum(h, 0.0)
    h = jnp.dot(h, w2_ref[...], preferred_element_type=jnp.float32) + b2_ref[...]
    h = jnp.maximum(h, 0.0)
    h = _bn_cols(h, g2_ref[...], bb2_ref[...])
    h = _bn_cols(h, go_ref[...], bo_ref[...])
    out_ref[...] = h


def _node_mlp(agg, x, eps, w1t, b1, g1, bb1, w2t, b2, g2, bb2, go, bo):
    return pl.pallas_call(
        _node_mlp_body,
        out_shape=jax.ShapeDtypeStruct((N, H), jnp.float32),
    )(agg, x, eps, w1t, b1, g1, bb1, w2t, b2, g2, bb2, go, bo)


def _readout_body(x1_ref, x2_ref, x3_ref, batch_ref, w1_ref, b1_ref,
                  g_ref, bb_ref, w2_ref, b2_ref, out_ref):
    bvec = batch_ref[...]
    gids = lax.broadcasted_iota(jnp.int32, (G, N), 0)
    onehot = (gids == bvec).astype(jnp.float32)
    p1 = jnp.dot(onehot, x1_ref[...], preferred_element_type=jnp.float32)
    p2 = jnp.dot(onehot, x2_ref[...], preferred_element_type=jnp.float32)
    p3 = jnp.dot(onehot, x3_ref[...], preferred_element_type=jnp.float32)
    h = jnp.concatenate([p1, p2, p3], axis=1)
    h = jnp.dot(h, w1_ref[...], preferred_element_type=jnp.float32) + b1_ref[...]
    h = _bn_cols(h, g_ref[...], bb_ref[...])
    h = jnp.where(h >= 0.0, h, 0.01 * h)
    out_ref[...] = (
        jnp.dot(h, w2_ref[...], preferred_element_type=jnp.float32) + b2_ref[...])


def _readout(x1, x2, x3, batch2d, w1t, b1, g, bb, w2t, b2):
    return pl.pallas_call(
        _readout_body,
        out_shape=jax.ShapeDtypeStruct((G, 1), jnp.float32),
    )(x1, x2, x3, batch2d, w1t, b1, g, bb, w2t, b2)


def kernel(x, edge_index, edge_attr, batch, params):
    p = params
    src = edge_index[0].astype(jnp.int32)
    dst = edge_index[1].astype(jnp.int32)

    w_cat = jnp.concatenate(
        [p['g1_lin_W'].T, p['g2_lin_W'].T, p['g3_lin_W'].T], axis=1)
    b_cat = jnp.concatenate(
        [p['g1_lin_b'], p['g2_lin_b'], p['g3_lin_b']]).reshape(1, 3 * H)
    e1, e2, e3 = _edge_linears(edge_attr.T, w_cat, b_cat)

    zeros = jnp.zeros((N, D), jnp.float32)

    def row(v):
        return v.reshape(1, -1)

    h = x
    feats = []
    for pre, e_l, og, ob in (
            ('g1_', e1, p['bn1_g'], p['bn1_b']),
            ('g2_', e2, p['bn2_g'], p['bn2_b']),
            ('g3_', e3, p['bn3_g'], p['bn3_b'])):
        agg = _sc_agg(h, src, dst, e_l, zeros)
        h = _node_mlp(
            agg, h, p[pre + 'eps'].reshape(1, 1).astype(jnp.float32),
            p[pre + 'W1'].T, row(p[pre + 'b1']),
            row(p[pre + 'bn1_g']), row(p[pre + 'bn1_b']),
            p[pre + 'W2'].T, row(p[pre + 'b2']),
            row(p[pre + 'bn2_g']), row(p[pre + 'bn2_b']),
            row(og), row(ob))
        feats.append(h)

    x1, x2, x3 = feats
    return _readout(
        x1, x2, x3, batch.astype(jnp.int32).reshape(1, N),
        p['lin1_W'].T, row(p['lin1_b']),
        row(p['bn4_g']), row(p['bn4_b']),
        p['lin2_W'].T, row(p['lin2_b']))


# speedup vs baseline: 5.7849x; 1.0236x over previous
import functools

import jax
import jax.numpy as jnp
from jax import lax
from jax.experimental import pallas as pl
from jax.experimental.pallas import tpu as pltpu
from jax.experimental.pallas import tpu_sc as plsc

N = 10000
E = 320000
D = 128
DE = 16
G = 64
H = 128

NC = 2
NS = 16
NW = NC * NS

EB = 64
NBLK = E // EB
BLK_LO = NBLK // NW
BLK_EXTRA = NBLK % NW
GROUP = 3
GROUPS = BLK_LO // GROUP
assert GROUPS * GROUP == BLK_LO
ROW_STRIDE = 624
ROW_COPY = 640


_EBLK = 6400


def _edge_lin_body(attr_ref, w_ref, b_ref, o_ref):
    h = lax.dot_general(attr_ref[...], w_ref[...],
                        dimension_numbers=(((0,), (0,)), ((), ())),
                        preferred_element_type=jnp.float32)
    o_ref[...] = h + b_ref[...]


def _edge_linear(attr_t, w, b):
    grid = (E // _EBLK,)
    return pl.pallas_call(
        _edge_lin_body,
        grid=grid,
        in_specs=[
            pl.BlockSpec((DE, _EBLK), lambda i: (0, i)),
            pl.BlockSpec((DE, H), lambda i: (0, 0)),
            pl.BlockSpec((1, H), lambda i: (0, 0)),
        ],
        out_specs=pl.BlockSpec((_EBLK, H), lambda i: (i, 0)),
        out_shape=jax.ShapeDtypeStruct((E, H), jnp.float32),
    )(attr_t, w, b)


_sc_mesh = plsc.VectorSubcoreMesh(
    core_axis_name="c", subcore_axis_name="s", num_cores=NC, num_subcores=NS)


@functools.partial(
    pl.kernel,
    out_type=jax.ShapeDtypeStruct((NC, N, D), jnp.float32),
    mesh=_sc_mesh,
    compiler_params=pltpu.CompilerParams(use_tc_tiling_on_sc=True),
    scratch_types=[
        pltpu.VMEM((EB,), jnp.int32),
        pltpu.VMEM((EB,), jnp.int32),
        pltpu.VMEM((EB,), jnp.int32),
        pltpu.VMEM((EB,), jnp.int32),
        pltpu.VMEM((EB,), jnp.int32),
        pltpu.VMEM((EB,), jnp.int32),
        pltpu.VMEM((EB, D), jnp.float32),
        pltpu.VMEM((EB, D), jnp.float32),
        pltpu.VMEM((EB, D), jnp.float32),
        pltpu.VMEM((EB, D), jnp.float32),
        pltpu.VMEM((EB, D), jnp.float32),
        pltpu.VMEM((EB, D), jnp.float32),
        pltpu.VMEM_SHARED((N, D), jnp.float32),
        pltpu.SemaphoreType.DMA,
        pltpu.SemaphoreType.DMA,
        pltpu.SemaphoreType.DMA,
        pltpu.SemaphoreType.DMA,
        pltpu.SemaphoreType.DMA,
        pltpu.SemaphoreType.DMA,
        pltpu.SemaphoreType.DMA,
        pltpu.SemaphoreType.DMA,
        pltpu.SemaphoreType.DMA,
        pltpu.SemaphoreType.DMA,
        pltpu.SemaphoreType.DMA,
        pltpu.SemaphoreType.DMA,
    ],
)
def _sc_agg(x_hbm, src_hbm, dst_hbm, e_hbm, zero_hbm, out_hbm,
            si0, si1, si2, di0, di1, di2, xg0, xg1, xg2, ev0, ev1, ev2,
            agg_sh, is0, is1, is2, gs0, gs1, gs2, es0, es1, es2,
            ss0, ss1, ss2):
    cid = lax.axis_index("c")
    sid = lax.axis_index("s")
    wid = sid * NC + cid
    row0 = sid * ROW_STRIDE
    nb = BLK_LO + jnp.where(wid < BLK_EXTRA, 1, 0)

    sis = [si0, si1, si2]
    dis = [di0, di1, di2]
    xgs = [xg0, xg1, xg2]
    evs = [ev0, ev1, ev2]
    isems = [is0, is1, is2]
    gsems = [gs0, gs1, gs2]
    esems = [es0, es1, es2]
    ssems = [ss0, ss1, ss2]

    def idx_start(i, sl):
        base = (wid + i * NW) * EB
        pltpu.make_async_copy(src_hbm.at[pl.ds(base, EB)], sis[sl],
                              isems[sl]).start()
        pltpu.make_async_copy(dst_hbm.at[pl.ds(base, EB)], dis[sl],
                              isems[sl]).start()

    def idx_wait(sl):
        pltpu.make_async_copy(src_hbm.at[pl.ds(0, EB)], sis[sl],
                              isems[sl]).wait()
        pltpu.make_async_copy(dst_hbm.at[pl.ds(0, EB)], dis[sl],
                              isems[sl]).wait()

    def e_start(i, sl):
        pltpu.make_async_copy(e_hbm.at[pl.ds((wid + i * NW) * EB, EB), :],
                              evs[sl], esems[sl]).start()

    def e_wait(sl):
        pltpu.make_async_copy(e_hbm.at[pl.ds(0, EB), :], evs[sl],
                              esems[sl]).wait()

    def g_start(sl_x, sl_i):
        pltpu.make_async_copy(x_hbm.at[sis[sl_i]], xgs[sl_x],
                              gsems[sl_x]).start()

    def g_wait(sl_x, sl_i):
        pltpu.make_async_copy(x_hbm.at[sis[sl_i]], xgs[sl_x],
                              gsems[sl_x]).wait()

    def s_start(sl):
        pltpu.async_copy(evs[sl], agg_sh.at[dis[sl]], ssems[sl], add=True)

    def s_wait(sl):
        pltpu.make_async_copy(evs[sl], agg_sh.at[dis[sl]], ssems[sl]).wait()

    def compute(sl_x, sl_e):
        xg = xgs[sl_x]
        ev = evs[sl_e]

        def row_body(r, c2):
            for rr in range(2):
                for cc in range(D // 16):
                    sl = pl.ds(cc * 16, 16)
                    ev[2 * r + rr, sl] = jnp.maximum(
                        xg[2 * r + rr, sl] + ev[2 * r + rr, sl], 0.0)
            return c2

        lax.fori_loop(0, EB // 2, row_body, 0)

    pltpu.sync_copy(zero_hbm.at[pl.ds(row0, ROW_COPY), :],
                    agg_sh.at[pl.ds(row0, ROW_COPY), :])
    plsc.subcore_barrier()

    idx_start(0, 0)
    idx_start(1, 1)
    idx_wait(0)
    g_start(0, 0)
    e_start(0, 0)
    e_start(1, 1)

    def group_body(g, carry):
        for jj in range(GROUP):
            i = g * GROUP + jj
            sl3 = jj % 3

            @pl.when(i >= 1)
            def _():
                s_wait((jj + 2) % 3)

            @pl.when(i + 2 < nb)
            def _():
                idx_start(i + 2, (jj + 2) % 3)
                e_start(i + 2, (jj + 2) % 3)

            @pl.when(i + 1 < nb)
            def _():
                idx_wait((jj + 1) % 3)
                g_start((jj + 1) % 3, (jj + 1) % 3)

            g_wait(sl3, sl3)
            e_wait(sl3)
            compute(sl3, sl3)
            s_start(sl3)
        return carry

    lax.fori_loop(0, GROUPS, group_body, 0)

    @pl.when(nb > BLK_LO)
    def _():
        s_wait(2)
        g_wait(0, 0)
        e_wait(0)
        compute(0, 0)
        s_start(0)
        s_wait(0)

    @pl.when(nb == BLK_LO)
    def _():
        s_wait(2)

    plsc.subcore_barrier()
    pltpu.sync_copy(agg_sh.at[pl.ds(row0, ROW_COPY), :],
                    out_hbm.at[cid, pl.ds(row0, ROW_COPY), :])


def _bn_cols(h, g, b):
    mu = jnp.mean(h, axis=0, keepdims=True)
    xc = h - mu
    var = jnp.mean(xc * xc, axis=0, keepdims=True)
    return g * xc * lax.rsqrt(var + 1e-5) + b


def _node_mlp_body(agg_ref, x_ref, eps_ref, w1_ref, b1_ref, g1_ref, bb1_ref,
                   w2_ref, b2_ref, g2_ref, bb2_ref, go_ref, bo_ref, out_ref):
    agg = agg_ref[0] + agg_ref[1]
    h0 = agg + (1.0 + eps_ref[0, 0]) * x_ref[...]
    h = jnp.dot(h0, w1_ref[...], preferred_element_type=jnp.float32) + b1_ref[...]
    h = _bn_cols(h, g1_ref[...], bb1_ref[...])
    h = jnp.maximum(h, 0.0)
    h = jnp.dot(h, w2_ref[...], preferred_element_type=jnp.float32) + b2_ref[...]
    h = jnp.maximum(h, 0.0)
    h = _bn_cols(h, g2_ref[...], bb2_ref[...])
    h = _bn_cols(h, go_ref[...], bo_ref[...])
    out_ref[...] = h


def _node_mlp(agg, x, eps, w1t, b1, g1, bb1, w2t, b2, g2, bb2, go, bo):
    return pl.pallas_call(
        _node_mlp_body,
        out_shape=jax.ShapeDtypeStruct((N, H), jnp.float32),
    )(agg, x, eps, w1t, b1, g1, bb1, w2t, b2, g2, bb2, go, bo)


def _readout_body(x1_ref, x2_ref, x3_ref, batch_ref, w1_ref, b1_ref,
                  g_ref, bb_ref, w2_ref, b2_ref, out_ref):
    bvec = batch_ref[...]
    gids = lax.broadcasted_iota(jnp.int32, (G, N), 0)
    onehot = (gids == bvec).astype(jnp.float32)
    p1 = jnp.dot(onehot, x1_ref[...], preferred_element_type=jnp.float32)
    p2 = jnp.dot(onehot, x2_ref[...], preferred_element_type=jnp.float32)
    p3 = jnp.dot(onehot, x3_ref[...], preferred_element_type=jnp.float32)
    h = jnp.concatenate([p1, p2, p3], axis=1)
    h = jnp.dot(h, w1_ref[...], preferred_element_type=jnp.float32) + b1_ref[...]
    h = _bn_cols(h, g_ref[...], bb_ref[...])
    h = jnp.where(h >= 0.0, h, 0.01 * h)
    out_ref[...] = (
        jnp.dot(h, w2_ref[...], preferred_element_type=jnp.float32) + b2_ref[...])


def _readout(x1, x2, x3, batch2d, w1t, b1, g, bb, w2t, b2):
    return pl.pallas_call(
        _readout_body,
        out_shape=jax.ShapeDtypeStruct((G, 1), jnp.float32),
    )(x1, x2, x3, batch2d, w1t, b1, g, bb, w2t, b2)


def kernel(x, edge_index, edge_attr, batch, params):
    p = params
    src = edge_index[0].astype(jnp.int32)
    dst = edge_index[1].astype(jnp.int32)

    zeros = jnp.zeros((N, D), jnp.float32)
    attr_t = edge_attr.T

    def row(v):
        return v.reshape(1, -1)

    h = x
    feats = []
    e_next = _edge_linear(attr_t, p['g1_lin_W'].T, row(p['g1_lin_b']))
    for pre, nxt, og, ob in (
            ('g1_', 'g2_', p['bn1_g'], p['bn1_b']),
            ('g2_', 'g3_', p['bn2_g'], p['bn2_b']),
            ('g3_', None, p['bn3_g'], p['bn3_b'])):
        agg = _sc_agg(h, src, dst, e_next, zeros)
        if nxt is not None:
            e_next = _edge_linear(attr_t, p[nxt + 'lin_W'].T,
                                  row(p[nxt + 'lin_b']))
        h = _node_mlp(
            agg, h, p[pre + 'eps'].reshape(1, 1).astype(jnp.float32),
            p[pre + 'W1'].T, row(p[pre + 'b1']),
            row(p[pre + 'bn1_g']), row(p[pre + 'bn1_b']),
            p[pre + 'W2'].T, row(p[pre + 'b2']),
            row(p[pre + 'bn2_g']), row(p[pre + 'bn2_b']),
            row(og), row(ob))
        feats.append(h)

    x1, x2, x3 = feats
    return _readout(
        x1, x2, x3, batch.astype(jnp.int32).reshape(1, N),
        p['lin1_W'].T, row(p['lin1_b']),
        row(p['bn4_g']), row(p['bn4_b']),
        p['lin2_W'].T, row(p['lin2_b']))
